# baseline (speedup 1.0000x reference)
import jax
import jax.numpy as jnp
from jax import lax
from jax.experimental import pallas as pl

_B, _N, _C = 128, 1024, 96
_NCLS = 1000
_K = 256
_BB = 64


def _select_weights(r):
    kk = jnp.int32(_K)
    ib = lax.bitcast_convert_type(r, jnp.int32)
    key = jnp.where(ib < 0, ib ^ jnp.int32(0x7FFFFFFF), ib)

    cnt0 = jnp.sum((key >= 0).astype(jnp.int32), axis=1, keepdims=True)
    prefix = jnp.where(cnt0 >= kk, jnp.int32(0), jnp.int32(-2147483648))

    def step(j, p):
        bit = jnp.int32(1) << (jnp.int32(30) - j)
        cand = p | bit
        cnt = jnp.sum((key >= cand).astype(jnp.int32), axis=1, keepdims=True)
        return jnp.where(cnt >= kk, cand, p)

    t = lax.fori_loop(0, 31, step, prefix)

    gt = key > t
    tie = key == t
    n_gt = jnp.sum(gt.astype(jnp.int32), axis=1, keepdims=True)
    need = kk - n_gt

    idx = lax.broadcasted_iota(jnp.int32, r.shape, 1)
    key2 = jnp.where(tie, jnp.int32(_N - 1) - idx, jnp.int32(-1))
    p2 = jnp.zeros_like(need)

    def step2(j, p):
        bit = jnp.int32(1) << (jnp.int32(9) - j)
        cand = p | bit
        cnt = jnp.sum((key2 >= cand).astype(jnp.int32), axis=1, keepdims=True)
        return jnp.where(cnt >= need, cand, p)

    p2 = lax.fori_loop(0, 10, step2, p2)
    sel = gt | (key2 >= p2)

    m = jnp.max(r, axis=1, keepdims=True)
    e = jnp.where(sel, jnp.exp(r - m), jnp.float32(0))
    z = jnp.sum(e, axis=1, keepdims=True)
    return e / z


_NQ = 4
_QB = _B // _NQ


def _body(rel_ref, t0, t1, t2, t3, fcw_ref, fcb_ref, out_ref):
    w = _select_weights(rel_ref[...])
    feats = []
    for q, tq in enumerate((t0, t1, t2, t3)):
        x = tq[...]
        wq = w[q * _QB:(q + 1) * _QB, :]
        feats.append(jnp.sum(x * wq[:, None, :], axis=2))
    feat = jnp.concatenate(feats, axis=0)
    logits = lax.dot_general(
        feat, fcw_ref[...], (((1,), (1,)), ((), ())),
        preferred_element_type=jnp.float32)
    out_ref[...] = logits + fcb_ref[...]


def kernel(tokens, reliability, fc_w, fc_b):
    tokens_t = jnp.transpose(tokens, (0, 2, 1))
    quarters = [lax.slice_in_dim(tokens_t, q * _QB, (q + 1) * _QB, axis=0)
                for q in range(_NQ)]
    fcb2 = fc_b.reshape(1, _NCLS)
    return pl.pallas_call(
        _body,
        grid=(1,),
        in_specs=[
            pl.BlockSpec((_B, _N), lambda i: (0, 0)),
            *[pl.BlockSpec((_QB, _C, _N), lambda i: (0, 0, 0))
              for _ in range(_NQ)],
            pl.BlockSpec((_NCLS, _C), lambda i: (0, 0)),
            pl.BlockSpec((1, _NCLS), lambda i: (0, 0)),
        ],
        out_specs=pl.BlockSpec((_B, _NCLS), lambda i: (0, 0)),
        out_shape=jax.ShapeDtypeStruct((_B, _NCLS), jnp.float32),
    )(reliability, *quarters, fc_w, fcb2)


# speedup vs baseline: 2.0462x; 2.0462x over previous
import jax
import jax.numpy as jnp
from jax import lax
from jax.experimental import pallas as pl

_B, _N, _C = 128, 1024, 96
_NCLS = 1000
_K = 256
_BB = 128


def _select_weights(r):
    kk = jnp.int32(_K)
    ib = lax.bitcast_convert_type(r, jnp.int32)
    key = jnp.where(ib < 0, ib ^ jnp.int32(0x7FFFFFFF), ib)

    cnt0 = jnp.sum((key >= 0).astype(jnp.int32), axis=1, keepdims=True)
    prefix = jnp.where(cnt0 >= kk, jnp.int32(0), jnp.int32(-2147483648))

    def step(j, p):
        bit = jnp.int32(1) << (jnp.int32(30) - j)
        cand = p | bit
        cnt = jnp.sum((key >= cand).astype(jnp.int32), axis=1, keepdims=True)
        return jnp.where(cnt >= kk, cand, p)

    t = lax.fori_loop(0, 31, step, prefix)

    gt = key > t
    tie = key == t
    n_gt = jnp.sum(gt.astype(jnp.int32), axis=1, keepdims=True)
    need = kk - n_gt

    idx = lax.broadcasted_iota(jnp.int32, r.shape, 1)
    key2 = jnp.where(tie, jnp.int32(_N - 1) - idx, jnp.int32(-1))
    p2 = jnp.zeros_like(need)

    def step2(j, p):
        bit = jnp.int32(1) << (jnp.int32(9) - j)
        cand = p | bit
        cnt = jnp.sum((key2 >= cand).astype(jnp.int32), axis=1, keepdims=True)
        return jnp.where(cnt >= need, cand, p)

    p2 = lax.fori_loop(0, 10, step2, p2)
    sel = gt | (key2 >= p2)

    m = jnp.max(r, axis=1, keepdims=True)
    e = jnp.where(sel, jnp.exp(r - m), jnp.float32(0))
    z = jnp.sum(e, axis=1, keepdims=True)
    return e / z


def _body(rel_ref, tokt_ref, fcw_ref, fcb_ref, out_ref):
    w = _select_weights(rel_ref[...])
    x = tokt_ref[...]
    feat = jnp.sum(x * w[:, None, :], axis=2)
    logits = lax.dot_general(
        feat, fcw_ref[...], (((1,), (1,)), ((), ())),
        preferred_element_type=jnp.float32)
    out_ref[...] = logits + fcb_ref[...]


def kernel(tokens, reliability, fc_w, fc_b):
    tokens_t = jnp.transpose(tokens, (0, 2, 1))
    fcb2 = fc_b.reshape(1, _NCLS)
    return pl.pallas_call(
        _body,
        grid=(_B // _BB,),
        in_specs=[
            pl.BlockSpec((_BB, _N), lambda i: (i, 0)),
            pl.BlockSpec((_BB, _C, _N), lambda i: (i, 0, 0)),
            pl.BlockSpec((_NCLS, _C), lambda i: (0, 0)),
            pl.BlockSpec((1, _NCLS), lambda i: (0, 0)),
        ],
        out_specs=pl.BlockSpec((_BB, _NCLS), lambda i: (i, 0)),
        out_shape=jax.ShapeDtypeStruct((_B, _NCLS), jnp.float32),
    )(reliability, tokens_t, fc_w, fcb2)


# speedup vs baseline: 2.2649x; 1.1069x over previous
import jax
import jax.numpy as jnp
from jax import lax
from jax.experimental import pallas as pl

_B, _N, _C = 128, 1024, 96
_NCLS = 1000
_K = 256
_BB = 64


def _select_weights(r):
    kk = jnp.int32(_K)
    ib = lax.bitcast_convert_type(r, jnp.int32)
    key = jnp.where(ib < 0, ib ^ jnp.int32(0x7FFFFFFF), ib)

    cnt0 = jnp.sum((key >= 0).astype(jnp.int32), axis=1, keepdims=True)
    prefix = jnp.where(cnt0 >= kk, jnp.int32(0), jnp.int32(-2147483648))

    def step(j, p):
        bit = jnp.int32(1) << (jnp.int32(30) - j)
        cand = p | bit
        cnt = jnp.sum((key >= cand).astype(jnp.int32), axis=1, keepdims=True)
        return jnp.where(cnt >= kk, cand, p)

    t = lax.fori_loop(0, 31, step, prefix)

    gt = key > t
    tie = key == t
    n_gt = jnp.sum(gt.astype(jnp.int32), axis=1, keepdims=True)
    need = kk - n_gt

    idx = lax.broadcasted_iota(jnp.int32, r.shape, 1)
    key2 = jnp.where(tie, jnp.int32(_N - 1) - idx, jnp.int32(-1))
    p2 = jnp.zeros_like(need)

    def step2(j, p):
        bit = jnp.int32(1) << (jnp.int32(9) - j)
        cand = p | bit
        cnt = jnp.sum((key2 >= cand).astype(jnp.int32), axis=1, keepdims=True)
        return jnp.where(cnt >= need, cand, p)

    p2 = lax.fori_loop(0, 10, step2, p2)
    sel = gt | (key2 >= p2)

    m = jnp.max(r, axis=1, keepdims=True)
    e = jnp.where(sel, jnp.exp(r - m), jnp.float32(0))
    z = jnp.sum(e, axis=1, keepdims=True)
    return e / z


def _body(rel_ref, tokt_ref, fcw_ref, fcb_ref, out_ref):
    w = _select_weights(rel_ref[...])
    x = tokt_ref[...]
    feat = jnp.sum(x * w[:, None, :], axis=2)
    logits = lax.dot_general(
        feat, fcw_ref[...], (((1,), (1,)), ((), ())),
        preferred_element_type=jnp.float32)
    out_ref[...] = logits + fcb_ref[...]


def kernel(tokens, reliability, fc_w, fc_b):
    tokens_t = jnp.transpose(tokens, (0, 2, 1))
    fcb2 = fc_b.reshape(1, _NCLS)
    return pl.pallas_call(
        _body,
        grid=(_B // _BB,),
        in_specs=[
            pl.BlockSpec((_BB, _N), lambda i: (i, 0)),
            pl.BlockSpec((_BB, _C, _N), lambda i: (i, 0, 0)),
            pl.BlockSpec((_NCLS, _C), lambda i: (0, 0)),
            pl.BlockSpec((1, _NCLS), lambda i: (0, 0)),
        ],
        out_specs=pl.BlockSpec((_BB, _NCLS), lambda i: (i, 0)),
        out_shape=jax.ShapeDtypeStruct((_B, _NCLS), jnp.float32),
    )(reliability, tokens_t, fc_w, fcb2)
